# Initial kernel scaffold; baseline (speedup 1.0000x reference)
import jax
import jax.numpy as jnp
from jax.experimental import pallas as pl


def kernel(pos, zinc_pos, edge_index, rbf_centers, rbf_widths, sh_w, sh_b, w1, b1, w2, b2, wz, bz, wn, bn, gamma, beta):
    raise NotImplementedError("write your pallas kernel here")


# speedup vs baseline: 5.4626x; 5.4626x over previous
import dataclasses
import math

import jax
import jax.numpy as jnp
from jax import lax
from jax.experimental import pallas as pl
from jax.experimental.pallas import tpu as pltpu
from jax.experimental.pallas import tpu_sc as plsc

N_NODES = 50000
NB = 16
H = 64
OUT = 128

E_PAD = 819200
EPT = E_PAD // 16
GW = 2048
SK = 512
SJ = SK // 128
BE = 2048
N_OUT = 51200
N_ACC = N_OUT
BN = 2048

_C0 = 1.0 / math.sqrt(4 * math.pi)
_C1 = math.sqrt(3 / (4 * math.pi))
_C2 = math.sqrt(15 / (4 * math.pi))
_C3 = math.sqrt(5 / (16 * math.pi))
_C4 = math.sqrt(15 / (16 * math.pi))


def _sc_compiler_params():
    cp = pltpu.CompilerParams(use_tc_tiling_on_sc=False)
    if "needs_layout_passes" in pltpu.CompilerParams.__dataclass_fields__:
        cp = dataclasses.replace(cp, needs_layout_passes=False)
    return cp


def _gather_windows(base, src_hbm, dst_hbm, idx_s, idx_d, comps):
    @pl.loop(0, EPT, step=GW)
    def _window(off):
        o = pl.multiple_of(base + off, GW)
        pltpu.sync_copy(src_hbm.at[pl.ds(o, GW)], idx_s)
        pltpu.sync_copy(dst_hbm.at[pl.ds(o, GW)], idx_d)

        @pl.loop(0, GW, step=16)
        def _grp(g):
            isv = idx_s[pl.ds(g, 16)]
            idv = idx_d[pl.ds(g, 16)]
            for tab, ow, _ in comps:
                ow[pl.ds(g, 16)] = (plsc.load_gather(tab, [idv])
                                    - plsc.load_gather(tab, [isv]))

        for _, ow, out_hbm in comps:
            pltpu.sync_copy(ow, out_hbm.at[pl.ds(o, GW)])


def _sc_gather_body(px_hbm, py_hbm, pz_hbm, src_hbm, dst_hbm,
                    rx_hbm, ry_hbm, rz_hbm,
                    tab0, tab1, idx_s, idx_d, o0, o1):
    c = lax.axis_index("c")
    s = lax.axis_index("s")
    base = s * EPT

    @pl.when(c == 0)
    def _():
        pltpu.sync_copy(px_hbm, tab0)
        pltpu.sync_copy(py_hbm, tab1)
        _gather_windows(base, src_hbm, dst_hbm, idx_s, idx_d,
                        [(tab0, o0, rx_hbm), (tab1, o1, ry_hbm)])

    @pl.when(c == 1)
    def _():
        pltpu.sync_copy(pz_hbm, tab0)
        _gather_windows(base, src_hbm, dst_hbm, idx_s, idx_d,
                        [(tab0, o0, rz_hbm)])


def _sc_scatter_body(pf_hbm, srcs_hbm, zero_hbm, agg_hbm, acc, idx_v, pf_v):
    c = lax.axis_index("c")
    s = lax.axis_index("s")
    zrows = N_ACC // 16
    zoff = pl.multiple_of(s * zrows, 8)
    pltpu.sync_copy(zero_hbm.at[pl.ds(zoff, zrows)],
                    acc.at[pl.ds(zoff, zrows)])
    plsc.subcore_barrier()

    @pl.loop(0, EPT, step=SK)
    def _window(off):
        o = pl.multiple_of(s * EPT + off, SK)
        pltpu.sync_copy(srcs_hbm.at[pl.ds(pl.multiple_of(o // 128, SJ), SJ)],
                        idx_v)
        pltpu.sync_copy(pf_hbm.at[c].at[pl.ds(o, SK)], pf_v)
        for j in range(SJ):
            pltpu.sync_copy(pf_v.at[pl.ds(j * 128, 128)],
                            acc.at[idx_v.at[j]], add=True)

    plsc.subcore_barrier()
    orows = N_OUT // 16
    ooff = pl.multiple_of(s * orows, 8)
    pltpu.sync_copy(acc.at[pl.ds(ooff, orows)],
                    agg_hbm.at[c].at[pl.ds(ooff, orows)])


def _edge_features(rx, ry, rz, cen, wid, shT, shb):
    d2 = rx * rx + ry * ry + rz * rz
    d = jnp.sqrt(d2)
    dc = jnp.maximum(d, 1e-6)
    inv = 1.0 / dc
    ux, uy, uz = rx * inv, ry * inv, rz * inv
    rbf = jnp.exp(-jnp.abs(wid) * (dc - cen) ** 2)
    one = jnp.ones_like(ux)
    sh = jnp.concatenate([
        _C0 * one,
        _C1 * uy,
        _C1 * uz,
        _C1 * ux,
        _C2 * ux * uy,
        _C2 * uy * uz,
        _C3 * (2.0 * uz * uz - ux * ux - uy * uy),
        _C2 * ux * uz,
        _C4 * (ux * ux - uy * uy),
    ], axis=0)
    shout = jnp.dot(shT, sh, preferred_element_type=jnp.float32) + shb
    return jnp.concatenate([rbf, shout], axis=0)


def _silu(x):
    return x * jax.nn.sigmoid(x)


def _tc_edge_body(rx_ref, ry_ref, rz_ref, cen_ref, wid_ref, shT_ref, shb_ref,
                  w1T_ref, b1_ref, w2T_ref, b2_ref, out_ref):
    rx = rx_ref[...].reshape(1, BE)
    ry = ry_ref[...].reshape(1, BE)
    rz = rz_ref[...].reshape(1, BE)
    feat = _edge_features(rx, ry, rz, cen_ref[...], wid_ref[...],
                          shT_ref[...], shb_ref[...])
    h1 = jnp.dot(w1T_ref[...], feat, preferred_element_type=jnp.float32)
    h1 = _silu(h1 + b1_ref[...])
    pf = jnp.dot(w2T_ref[...], h1, preferred_element_type=jnp.float32)
    pf = pf + b2_ref[...]
    pfT = pf.T
    q = BE // 4
    out_ref[0] = jnp.concatenate(
        [pfT[k * q:(k + 1) * q, :32] for k in range(4)], axis=1)
    out_ref[1] = jnp.concatenate(
        [pfT[k * q:(k + 1) * q, 32:] for k in range(4)], axis=1)


def _tc_node_body(posT_ref, zinc_ref, cen_ref, wid_ref, shT_ref, shb_ref,
                  wzT_ref, bz_ref, agg_ref, wn0_ref, wn1_ref, wn2_ref,
                  bn_ref, g_ref, b_ref, out_ref):
    p = posT_ref[...] - zinc_ref[...]
    feat = _edge_features(p[0:1], p[1:2], p[2:3], cen_ref[...], wid_ref[...],
                          shT_ref[...], shb_ref[...])
    zn = jnp.dot(wzT_ref[...], feat, preferred_element_type=jnp.float32)
    zn = _silu(zn + bz_ref[...])
    znT = zn.T
    a0 = jnp.concatenate(
        [agg_ref[0][:, 32 * k:32 * (k + 1)] for k in range(4)], axis=0)
    a1 = jnp.concatenate(
        [agg_ref[1][:, 32 * k:32 * (k + 1)] for k in range(4)], axis=0)
    h = (jnp.dot(a0, wn0_ref[...], preferred_element_type=jnp.float32)
         + jnp.dot(a1, wn1_ref[...], preferred_element_type=jnp.float32)
         + jnp.dot(znT, wn2_ref[...], preferred_element_type=jnp.float32)
         + bn_ref[...])
    m = jnp.mean(h, axis=-1, keepdims=True)
    hc = h - m
    v = jnp.mean(hc * hc, axis=-1, keepdims=True)
    y = hc / jnp.sqrt(v + 1e-5) * g_ref[...] + b_ref[...]
    out_ref[...] = _silu(y)


def _full(shape):
    return pl.BlockSpec(shape, lambda i: tuple(0 for _ in shape))


def kernel(pos, zinc_pos, edge_index, rbf_centers, rbf_widths, sh_w, sh_b,
           w1, b1, w2, b2, wz, bz, wn, bn, gamma, beta):
    E = edge_index.shape[1]
    pad = E_PAD - E
    src = edge_index[0].astype(jnp.int32)
    dst = edge_index[1].astype(jnp.int32)
    src_g = jnp.concatenate([src, jnp.zeros((pad,), jnp.int32)])
    dst_g = jnp.concatenate([dst, jnp.zeros((pad,), jnp.int32)])
    dummy = N_NODES + (jnp.arange(pad, dtype=jnp.int32) % 1024)
    src_s = (jnp.concatenate([src, dummy])
             .reshape(E_PAD // BE, 4, BE // 4)
             .swapaxes(1, 2)
             .reshape(-1, 128))
    pos_x = pos[:, 0]
    pos_y = pos[:, 1]
    pos_z = pos[:, 2]
    posT = jnp.pad(pos.T, ((0, 0), (0, N_OUT - N_NODES)))
    posTp = (posT.reshape(3, N_OUT // BN, BN // 4, 4)
             .swapaxes(2, 3)
             .reshape(3, N_OUT))

    cen = rbf_centers.reshape(NB, 1)
    wid = rbf_widths.reshape(NB, 1)
    shT = sh_w.T
    shb = sh_b.reshape(NB, 1)
    w1T = w1.T
    b1c = b1.reshape(H, 1)
    w2T = w2.T
    b2c = b2.reshape(H, 1)
    wzT = wz.T
    bzc = bz.reshape(H, 1)
    wn0 = wn[:32]
    wn1 = wn[32:64]
    wn2 = wn[64:]
    bnr = bn.reshape(1, OUT)
    gr = gamma.reshape(1, OUT)
    br = beta.reshape(1, OUT)
    zinc_col = zinc_pos.reshape(3, 1)
    zeros_acc = jnp.zeros((N_ACC, 32), jnp.float32)

    mesh = plsc.VectorSubcoreMesh(core_axis_name="c", subcore_axis_name="s")
    cp = _sc_compiler_params()

    rx, ry, rz = pl.kernel(
        _sc_gather_body,
        out_type=[jax.ShapeDtypeStruct((E_PAD,), jnp.float32)] * 3,
        mesh=mesh,
        compiler_params=cp,
        scratch_types=[
            pltpu.VMEM((N_NODES,), jnp.float32),
            pltpu.VMEM((N_NODES,), jnp.float32),
            pltpu.VMEM((GW,), jnp.int32),
            pltpu.VMEM((GW,), jnp.int32),
            pltpu.VMEM((GW,), jnp.float32),
            pltpu.VMEM((GW,), jnp.float32),
        ],
    )(pos_x, pos_y, pos_z, src_g, dst_g)

    nbe = E_PAD // BE
    rx3 = rx.reshape(nbe, 1, BE)
    ry3 = ry.reshape(nbe, 1, BE)
    rz3 = rz.reshape(nbe, 1, BE)

    pf4 = pl.pallas_call(
        _tc_edge_body,
        grid=(nbe,),
        in_specs=[
            pl.BlockSpec((1, 1, BE), lambda i: (i, 0, 0)),
            pl.BlockSpec((1, 1, BE), lambda i: (i, 0, 0)),
            pl.BlockSpec((1, 1, BE), lambda i: (i, 0, 0)),
            _full((NB, 1)),
            _full((NB, 1)),
            _full((NB, 9)),
            _full((NB, 1)),
            _full((H, 2 * NB)),
            _full((H, 1)),
            _full((H, H)),
            _full((H, 1)),
        ],
        out_specs=pl.BlockSpec((2, BE // 4, 128), lambda i: (0, i, 0)),
        out_shape=jax.ShapeDtypeStruct((2, E_PAD // 4, 128), jnp.float32),
    )(rx3, ry3, rz3, cen, wid, shT, shb, w1T, b1c, w2T, b2c)

    pf3 = pf4.reshape(2, E_PAD, 32)

    agg2 = pl.kernel(
        _sc_scatter_body,
        out_type=jax.ShapeDtypeStruct((2, N_OUT, 32), jnp.float32),
        mesh=mesh,
        compiler_params=cp,
        scratch_types=[
            pltpu.VMEM_SHARED((N_ACC, 32), jnp.float32),
            pltpu.VMEM((SJ, 128), jnp.int32),
            pltpu.VMEM((SK, 32), jnp.float32),
        ],
    )(pf3, src_s, zeros_acc)

    agg4 = agg2.reshape(2, N_OUT // 4, 128)

    out_pad = pl.pallas_call(
        _tc_node_body,
        grid=(N_OUT // BN,),
        in_specs=[
            pl.BlockSpec((3, BN), lambda i: (0, i)),
            _full((3, 1)),
            _full((NB, 1)),
            _full((NB, 1)),
            _full((NB, 9)),
            _full((NB, 1)),
            _full((H, 2 * NB)),
            _full((H, 1)),
            pl.BlockSpec((2, BN // 4, 128), lambda i: (0, i, 0)),
            _full((32, OUT)),
            _full((32, OUT)),
            _full((H, OUT)),
            _full((1, OUT)),
            _full((1, OUT)),
            _full((1, OUT)),
        ],
        out_specs=pl.BlockSpec((BN, OUT), lambda i: (i, 0)),
        out_shape=jax.ShapeDtypeStruct((N_OUT, OUT), jnp.float32),
    )(posTp, zinc_col, cen, wid, shT, shb, wzT, bzc, agg4,
      wn0, wn1, wn2, bnr, gr, br)

    out = (out_pad.reshape(N_OUT // BN, 4, BN // 4, OUT)
           .swapaxes(1, 2)
           .reshape(N_OUT, OUT))
    return out[:N_NODES]


# speedup vs baseline: 6.0908x; 1.1150x over previous
import dataclasses
import math

import jax
import jax.numpy as jnp
from jax import lax
from jax.experimental import pallas as pl
from jax.experimental.pallas import tpu as pltpu
from jax.experimental.pallas import tpu_sc as plsc

N_NODES = 50000
NB = 16
H = 64
OUT = 128

E_PAD = 819200
EPT = E_PAD // 16
GW = 2048
SK = 512
SJ = SK // 128
BE = 2048
N_OUT = 51200
N_ACC = N_OUT
BN = 2048

_C0 = 1.0 / math.sqrt(4 * math.pi)
_C1 = math.sqrt(3 / (4 * math.pi))
_C2 = math.sqrt(15 / (4 * math.pi))
_C3 = math.sqrt(5 / (16 * math.pi))
_C4 = math.sqrt(15 / (16 * math.pi))


def _sc_compiler_params():
    cp = pltpu.CompilerParams(use_tc_tiling_on_sc=False)
    if "needs_layout_passes" in pltpu.CompilerParams.__dataclass_fields__:
        cp = dataclasses.replace(cp, needs_layout_passes=False)
    return cp


def _gather_windows(base, src_hbm, dst_hbm, idx_s, idx_d, comps):
    @pl.loop(0, EPT, step=GW)
    def _window(off):
        o = pl.multiple_of(base + off, GW)
        pltpu.sync_copy(src_hbm.at[pl.ds(o, GW)], idx_s)
        pltpu.sync_copy(dst_hbm.at[pl.ds(o, GW)], idx_d)

        @pl.loop(0, GW, step=16)
        def _grp(g):
            isv = idx_s[pl.ds(g, 16)]
            idv = idx_d[pl.ds(g, 16)]
            for tab, ow, _ in comps:
                ow[pl.ds(g, 16)] = (plsc.load_gather(tab, [idv])
                                    - plsc.load_gather(tab, [isv]))

        for _, ow, out_hbm in comps:
            pltpu.sync_copy(ow, out_hbm.at[pl.ds(o, GW)])


def _sc_gather_body(px_hbm, py_hbm, pz_hbm, src_hbm, dst_hbm,
                    rx_hbm, ry_hbm, rz_hbm,
                    tab0, tab1, idx_s, idx_d, o0, o1):
    c = lax.axis_index("c")
    s = lax.axis_index("s")
    base = s * EPT

    @pl.when(c == 0)
    def _():
        pltpu.sync_copy(px_hbm, tab0)
        pltpu.sync_copy(py_hbm, tab1)
        _gather_windows(base, src_hbm, dst_hbm, idx_s, idx_d,
                        [(tab0, o0, rx_hbm), (tab1, o1, ry_hbm)])

    @pl.when(c == 1)
    def _():
        pltpu.sync_copy(pz_hbm, tab0)
        _gather_windows(base, src_hbm, dst_hbm, idx_s, idx_d,
                        [(tab0, o0, rz_hbm)])


def _sc_scatter_body(pf_hbm, srcs_hbm, zero_hbm, agg_hbm, acc, idx_v, pf_v):
    c = lax.axis_index("c")
    s = lax.axis_index("s")
    zrows = N_ACC // 16
    zoff = pl.multiple_of(s * zrows, 8)
    pltpu.sync_copy(zero_hbm.at[pl.ds(zoff, zrows)],
                    acc.at[pl.ds(zoff, zrows)])
    plsc.subcore_barrier()

    @pl.loop(0, EPT, step=SK)
    def _window(off):
        o = pl.multiple_of(s * EPT + off, SK)
        pltpu.sync_copy(srcs_hbm.at[pl.ds(pl.multiple_of(o // 128, SJ), SJ)],
                        idx_v)
        pltpu.sync_copy(pf_hbm.at[c].at[pl.ds(o, SK)], pf_v)
        for j in range(SJ):
            pltpu.sync_copy(pf_v.at[pl.ds(j * 128, 128)],
                            acc.at[idx_v.at[j]], add=True)

    plsc.subcore_barrier()
    orows = N_OUT // 16
    ooff = pl.multiple_of(s * orows, 8)
    pltpu.sync_copy(acc.at[pl.ds(ooff, orows)],
                    agg_hbm.at[c].at[pl.ds(ooff, orows)])


def _silu(x):
    return x * (1.0 / (1.0 + jnp.exp(-x)))


def _hidden(rx, ry, rz, cen, wid, wf, bf):
    d2 = rx * rx + ry * ry + rz * rz
    d = jnp.sqrt(d2)
    dc = jnp.maximum(d, 1e-6)
    inv = 1.0 / dc
    ux, uy, uz = rx * inv, ry * inv, rz * inv
    one = jnp.ones_like(ux)
    rbf = jnp.exp(-jnp.abs(wid) * (dc - cen) ** 2)
    g = jnp.concatenate([
        rbf,
        _C0 * one,
        _C1 * uy,
        _C1 * uz,
        _C1 * ux,
        _C2 * ux * uy,
        _C2 * uy * uz,
        _C3 * (2.0 * uz * uz - ux * ux - uy * uy),
        _C2 * ux * uz,
        _C4 * (ux * ux - uy * uy),
    ], axis=0).astype(jnp.bfloat16)
    h = jnp.dot(wf, g, preferred_element_type=jnp.float32) + bf
    return _silu(h)


def _dotT(a, b):
    return jax.lax.dot_general(a, b, (((0,), (0,)), ((), ())),
                               preferred_element_type=jnp.float32)


def _tc_edge_body(rx_ref, ry_ref, rz_ref, cen_ref, wid_ref,
                  wf1_ref, b1f_ref, w2_ref, b2_ref, out_ref):
    rx = rx_ref[...].reshape(1, BE)
    ry = ry_ref[...].reshape(1, BE)
    rz = rz_ref[...].reshape(1, BE)
    h1 = _hidden(rx, ry, rz, cen_ref[...], wid_ref[...],
                 wf1_ref[...], b1f_ref[...])
    pf = _dotT(h1.astype(jnp.bfloat16), w2_ref[...])
    pf = pf + b2_ref[...]
    q = BE // 4
    out_ref[0] = jnp.concatenate(
        [pf[k * q:(k + 1) * q, :32] for k in range(4)], axis=1)
    out_ref[1] = jnp.concatenate(
        [pf[k * q:(k + 1) * q, 32:] for k in range(4)], axis=1)


def _tc_node_body(posT_ref, zinc_ref, cen_ref, wid_ref,
                  wfz_ref, bzf_ref, agg_ref, wn0_ref, wn1_ref,
                  wn2_ref, bn_ref, g_ref, b_ref, out_ref):
    p = posT_ref[...] - zinc_ref[...]
    zn = _hidden(p[0:1], p[1:2], p[2:3], cen_ref[...], wid_ref[...],
                 wfz_ref[...], bzf_ref[...])
    a0 = jnp.concatenate(
        [agg_ref[0][:, 32 * k:32 * (k + 1)] for k in range(4)], axis=0)
    a1 = jnp.concatenate(
        [agg_ref[1][:, 32 * k:32 * (k + 1)] for k in range(4)], axis=0)
    h = (jnp.dot(a0, wn0_ref[...], preferred_element_type=jnp.float32)
         + jnp.dot(a1, wn1_ref[...], preferred_element_type=jnp.float32)
         + _dotT(zn, wn2_ref[...])
         + bn_ref[...])
    m = jnp.mean(h, axis=-1, keepdims=True)
    hc = h - m
    v = jnp.mean(hc * hc, axis=-1, keepdims=True)
    y = hc / jnp.sqrt(v + 1e-5) * g_ref[...] + b_ref[...]
    out_ref[...] = _silu(y)


def _full(shape):
    return pl.BlockSpec(shape, lambda i: tuple(0 for _ in shape))


def kernel(pos, zinc_pos, edge_index, rbf_centers, rbf_widths, sh_w, sh_b,
           w1, b1, w2, b2, wz, bz, wn, bn, gamma, beta):
    E = edge_index.shape[1]
    pad = E_PAD - E
    src = edge_index[0].astype(jnp.int32)
    dst = edge_index[1].astype(jnp.int32)
    src_g = jnp.concatenate([src, jnp.zeros((pad,), jnp.int32)])
    dst_g = jnp.concatenate([dst, jnp.zeros((pad,), jnp.int32)])
    dummy = N_NODES + (jnp.arange(pad, dtype=jnp.int32) % 1024)
    src_s = (jnp.concatenate([src, dummy])
             .reshape(E_PAD // BE, 4, BE // 4)
             .swapaxes(1, 2)
             .reshape(-1, 128))
    pos_x = pos[:, 0]
    pos_y = pos[:, 1]
    pos_z = pos[:, 2]
    posT = jnp.pad(pos.T, ((0, 0), (0, N_OUT - N_NODES)))
    posTp = (posT.reshape(3, N_OUT // BN, BN // 4, 4)
             .swapaxes(2, 3)
             .reshape(3, N_OUT))

    cen = rbf_centers.reshape(NB, 1)
    wid = rbf_widths.reshape(NB, 1)
    bf16 = jnp.bfloat16
    wf1 = jnp.concatenate([w1[:NB], sh_w @ w1[NB:]], axis=0).T.astype(bf16)
    b1f = (b1 + sh_b @ w1[NB:]).reshape(H, 1)
    wfz = jnp.concatenate([wz[:NB], sh_w @ wz[NB:]], axis=0).T.astype(bf16)
    bzf = (bz + sh_b @ wz[NB:]).reshape(H, 1)
    w2b = w2.astype(bf16)
    b2r = b2.reshape(1, H)
    wn0 = wn[:32]
    wn1 = wn[32:64]
    wn2 = wn[64:]
    bnr = bn.reshape(1, OUT)
    gr = gamma.reshape(1, OUT)
    br = beta.reshape(1, OUT)
    zinc_col = zinc_pos.reshape(3, 1)
    zeros_acc = jnp.zeros((N_ACC, 32), jnp.float32)

    mesh = plsc.VectorSubcoreMesh(core_axis_name="c", subcore_axis_name="s")
    cp = _sc_compiler_params()

    rx, ry, rz = pl.kernel(
        _sc_gather_body,
        out_type=[jax.ShapeDtypeStruct((E_PAD,), jnp.float32)] * 3,
        mesh=mesh,
        compiler_params=cp,
        scratch_types=[
            pltpu.VMEM((N_NODES,), jnp.float32),
            pltpu.VMEM((N_NODES,), jnp.float32),
            pltpu.VMEM((GW,), jnp.int32),
            pltpu.VMEM((GW,), jnp.int32),
            pltpu.VMEM((GW,), jnp.float32),
            pltpu.VMEM((GW,), jnp.float32),
        ],
    )(pos_x, pos_y, pos_z, src_g, dst_g)

    nbe = E_PAD // BE
    rx3 = rx.reshape(nbe, 1, BE)
    ry3 = ry.reshape(nbe, 1, BE)
    rz3 = rz.reshape(nbe, 1, BE)

    pf4 = pl.pallas_call(
        _tc_edge_body,
        grid=(nbe,),
        in_specs=[
            pl.BlockSpec((1, 1, BE), lambda i: (i, 0, 0)),
            pl.BlockSpec((1, 1, BE), lambda i: (i, 0, 0)),
            pl.BlockSpec((1, 1, BE), lambda i: (i, 0, 0)),
            _full((NB, 1)),
            _full((NB, 1)),
            _full((H, NB + 9)),
            _full((H, 1)),
            _full((H, H)),
            _full((1, H)),
        ],
        out_specs=pl.BlockSpec((2, BE // 4, 128), lambda i: (0, i, 0)),
        out_shape=jax.ShapeDtypeStruct((2, E_PAD // 4, 128), jnp.float32),
        compiler_params=pltpu.CompilerParams(
            fuse_transposed_lhs_in_matmul=True),
    )(rx3, ry3, rz3, cen, wid, wf1, b1f, w2b, b2r)

    pf3 = pf4.reshape(2, E_PAD, 32)

    agg2 = pl.kernel(
        _sc_scatter_body,
        out_type=jax.ShapeDtypeStruct((2, N_OUT, 32), jnp.float32),
        mesh=mesh,
        compiler_params=cp,
        scratch_types=[
            pltpu.VMEM_SHARED((N_ACC, 32), jnp.float32),
            pltpu.VMEM((SJ, 128), jnp.int32),
            pltpu.VMEM((SK, 32), jnp.float32),
        ],
    )(pf3, src_s, zeros_acc)

    agg4 = agg2.reshape(2, N_OUT // 4, 128)

    out_pad = pl.pallas_call(
        _tc_node_body,
        grid=(N_OUT // BN,),
        in_specs=[
            pl.BlockSpec((3, BN), lambda i: (0, i)),
            _full((3, 1)),
            _full((NB, 1)),
            _full((NB, 1)),
            _full((H, NB + 9)),
            _full((H, 1)),
            pl.BlockSpec((2, BN // 4, 128), lambda i: (0, i, 0)),
            _full((32, OUT)),
            _full((32, OUT)),
            _full((H, OUT)),
            _full((1, OUT)),
            _full((1, OUT)),
            _full((1, OUT)),
        ],
        out_specs=pl.BlockSpec((BN, OUT), lambda i: (i, 0)),
        out_shape=jax.ShapeDtypeStruct((N_OUT, OUT), jnp.float32),
        compiler_params=pltpu.CompilerParams(
            fuse_transposed_lhs_in_matmul=True),
    )(posTp, zinc_col, cen, wid, wfz, bzf, agg4,
      wn0, wn1, wn2, bnr, gr, br)

    out = (out_pad.reshape(N_OUT // BN, 4, BN // 4, OUT)
           .swapaxes(1, 2)
           .reshape(N_OUT, OUT))
    return out[:N_NODES]


# speedup vs baseline: 6.3233x; 1.0382x over previous
import dataclasses
import math

import jax
import jax.numpy as jnp
from jax import lax
from jax.experimental import pallas as pl
from jax.experimental.pallas import tpu as pltpu
from jax.experimental.pallas import tpu_sc as plsc

N_NODES = 50000
NB = 16
H = 64
OUT = 128

NCH = 2
E_PAD = 819200
E_CH = E_PAD // NCH
ECT = E_CH // 16
GW = 1600
SK = 512
SKJ = SK // 128
BE = 2048
N_OUT = 51200
N_ACC = N_OUT
BN = 2048

_C0 = 1.0 / math.sqrt(4 * math.pi)
_C1 = math.sqrt(3 / (4 * math.pi))
_C2 = math.sqrt(15 / (4 * math.pi))
_C3 = math.sqrt(5 / (16 * math.pi))
_C4 = math.sqrt(15 / (16 * math.pi))


def _sc_compiler_params():
    cp = pltpu.CompilerParams(use_tc_tiling_on_sc=False)
    if "needs_layout_passes" in pltpu.CompilerParams.__dataclass_fields__:
        cp = dataclasses.replace(cp, needs_layout_passes=False)
    return cp


def _gather_windows(base, src_hbm, dst_hbm, idx_s, idx_d, comps):
    @pl.loop(0, ECT, step=GW)
    def _window(off):
        o = pl.multiple_of(base + off, GW)
        pltpu.sync_copy(src_hbm.at[pl.ds(o, GW)], idx_s)
        pltpu.sync_copy(dst_hbm.at[pl.ds(o, GW)], idx_d)

        @pl.loop(0, GW, step=16)
        def _grp(g):
            isv = idx_s[pl.ds(g, 16)]
            idv = idx_d[pl.ds(g, 16)]
            for tab, ow, _ in comps:
                ow[pl.ds(g, 16)] = (plsc.load_gather(tab, [idv])
                                    - plsc.load_gather(tab, [isv]))

        for _, ow, out_hbm in comps:
            pltpu.sync_copy(ow, out_hbm.at[pl.ds(o, GW)])


def _sc_gather_body(px_hbm, py_hbm, pz_hbm, src_hbm, dst_hbm,
                    rx_hbm, ry_hbm, rz_hbm,
                    tab0, tab1, idx_s, idx_d, o0, o1):
    c = lax.axis_index("c")
    s = lax.axis_index("s")
    base = s * ECT

    @pl.when(c == 0)
    def _():
        pltpu.sync_copy(px_hbm, tab0)
        pltpu.sync_copy(py_hbm, tab1)
        _gather_windows(base, src_hbm, dst_hbm, idx_s, idx_d,
                        [(tab0, o0, rx_hbm), (tab1, o1, ry_hbm)])

    @pl.when(c == 1)
    def _():
        pltpu.sync_copy(pz_hbm, tab0)
        _gather_windows(base, src_hbm, dst_hbm, idx_s, idx_d,
                        [(tab0, o0, rz_hbm)])


def _sc_scatter_body(pf_hbm, srcs_hbm, zero_hbm, agg_hbm, acc, idx0, pfb0):
    c = lax.axis_index("c")
    s = lax.axis_index("s")
    zrows = N_ACC // 16
    zoff = pl.multiple_of(s * zrows, 8)
    pltpu.sync_copy(zero_hbm.at[pl.ds(zoff, zrows)],
                    acc.at[pl.ds(zoff, zrows)])
    plsc.subcore_barrier()

    base = s * ECT

    @pl.loop(0, ECT, step=SK)
    def _window(off):
        o = pl.multiple_of(base + off, SK)
        pltpu.sync_copy(
            srcs_hbm.at[pl.ds(pl.multiple_of(o // 128, SKJ), SKJ)], idx0)
        pltpu.sync_copy(pf_hbm.at[c].at[pl.ds(o, SK)], pfb0)
        for j in range(SKJ):
            pltpu.sync_copy(pfb0.at[pl.ds(j * 128, 128)],
                            acc.at[idx0.at[j]], add=True)

    plsc.subcore_barrier()
    orows = N_OUT // 16
    ooff = pl.multiple_of(s * orows, 8)
    pltpu.sync_copy(acc.at[pl.ds(ooff, orows)],
                    agg_hbm.at[c].at[pl.ds(ooff, orows)])


def _silu(x):
    return x * (1.0 / (1.0 + jnp.exp(-x)))


def _hidden(rx, ry, rz, cen, wid, wf, bf):
    d2 = rx * rx + ry * ry + rz * rz
    d = jnp.sqrt(d2)
    dc = jnp.maximum(d, 1e-6)
    inv = 1.0 / dc
    ux, uy, uz = rx * inv, ry * inv, rz * inv
    one = jnp.ones_like(ux)
    rbf = jnp.exp(-jnp.abs(wid) * (dc - cen) ** 2)
    g = jnp.concatenate([
        rbf,
        _C0 * one,
        _C1 * uy,
        _C1 * uz,
        _C1 * ux,
        _C2 * ux * uy,
        _C2 * uy * uz,
        _C3 * (2.0 * uz * uz - ux * ux - uy * uy),
        _C2 * ux * uz,
        _C4 * (ux * ux - uy * uy),
    ], axis=0).astype(jnp.bfloat16)
    h = jnp.dot(wf, g, preferred_element_type=jnp.float32) + bf
    return _silu(h)


def _dotT(a, b):
    return jax.lax.dot_general(a, b, (((0,), (0,)), ((), ())),
                               preferred_element_type=jnp.float32)


def _tc_edge_body(rx_ref, ry_ref, rz_ref, cen_ref, wid_ref,
                  wf1_ref, b1f_ref, w2_ref, b2_ref, out_ref):
    rx = rx_ref[...].reshape(1, BE)
    ry = ry_ref[...].reshape(1, BE)
    rz = rz_ref[...].reshape(1, BE)
    h1 = _hidden(rx, ry, rz, cen_ref[...], wid_ref[...],
                 wf1_ref[...], b1f_ref[...])
    pf = _dotT(h1.astype(jnp.bfloat16), w2_ref[...])
    pf = pf + b2_ref[...]
    q = BE // 4
    out_ref[0] = jnp.concatenate(
        [pf[k * q:(k + 1) * q, :32] for k in range(4)], axis=1)
    out_ref[1] = jnp.concatenate(
        [pf[k * q:(k + 1) * q, 32:] for k in range(4)], axis=1)


def _tc_node_body(posT_ref, zinc_ref, cen_ref, wid_ref,
                  wfz_ref, bzf_ref, agga_ref, aggb_ref, wn0_ref, wn1_ref,
                  wn2_ref, bn_ref, g_ref, b_ref, out_ref):
    p = posT_ref[...] - zinc_ref[...]
    zn = _hidden(p[0:1], p[1:2], p[2:3], cen_ref[...], wid_ref[...],
                 wfz_ref[...], bzf_ref[...])
    def unpack(ref, half):
        return jnp.concatenate(
            [ref[half][:, 32 * k:32 * (k + 1)] for k in range(4)], axis=0)

    a0 = unpack(agga_ref, 0) + unpack(aggb_ref, 0)
    a1 = unpack(agga_ref, 1) + unpack(aggb_ref, 1)
    h = (jnp.dot(a0, wn0_ref[...], preferred_element_type=jnp.float32)
         + jnp.dot(a1, wn1_ref[...], preferred_element_type=jnp.float32)
         + _dotT(zn, wn2_ref[...])
         + bn_ref[...])
    m = jnp.mean(h, axis=-1, keepdims=True)
    hc = h - m
    v = jnp.mean(hc * hc, axis=-1, keepdims=True)
    y = hc / jnp.sqrt(v + 1e-5) * g_ref[...] + b_ref[...]
    out_ref[...] = _silu(y)


def _full(shape):
    return pl.BlockSpec(shape, lambda i: tuple(0 for _ in shape))


def kernel(pos, zinc_pos, edge_index, rbf_centers, rbf_widths, sh_w, sh_b,
           w1, b1, w2, b2, wz, bz, wn, bn, gamma, beta):
    E = edge_index.shape[1]
    pad = E_PAD - E
    src = edge_index[0].astype(jnp.int32)
    dst = edge_index[1].astype(jnp.int32)
    src_g = jnp.concatenate([src, jnp.zeros((pad,), jnp.int32)])
    dst_g = jnp.concatenate([dst, jnp.zeros((pad,), jnp.int32)])
    dummy = N_NODES + (jnp.arange(pad, dtype=jnp.int32) % 1024)
    src_s = (jnp.concatenate([src, dummy])
             .reshape(E_PAD // BE, 4, BE // 4)
             .swapaxes(1, 2)
             .reshape(-1, 128))
    pos_x = pos[:, 0]
    pos_y = pos[:, 1]
    pos_z = pos[:, 2]
    posT = jnp.pad(pos.T, ((0, 0), (0, N_OUT - N_NODES)))
    posTp = (posT.reshape(3, N_OUT // BN, BN // 4, 4)
             .swapaxes(2, 3)
             .reshape(3, N_OUT))

    cen = rbf_centers.reshape(NB, 1)
    wid = rbf_widths.reshape(NB, 1)
    bf16 = jnp.bfloat16
    wf1 = jnp.concatenate([w1[:NB], sh_w @ w1[NB:]], axis=0).T.astype(bf16)
    b1f = (b1 + sh_b @ w1[NB:]).reshape(H, 1)
    wfz = jnp.concatenate([wz[:NB], sh_w @ wz[NB:]], axis=0).T.astype(bf16)
    bzf = (bz + sh_b @ wz[NB:]).reshape(H, 1)
    w2b = w2.astype(bf16)
    b2r = b2.reshape(1, H)
    wn0 = wn[:32]
    wn1 = wn[32:64]
    wn2 = wn[64:]
    bnr = bn.reshape(1, OUT)
    gr = gamma.reshape(1, OUT)
    br = beta.reshape(1, OUT)
    zinc_col = zinc_pos.reshape(3, 1)
    zeros_acc = jnp.zeros((N_ACC, 32), jnp.float32)

    mesh = plsc.VectorSubcoreMesh(core_axis_name="c", subcore_axis_name="s")
    cp = _sc_compiler_params()

    def sc_gather(src_c, dst_c):
        return pl.kernel(
            _sc_gather_body,
            out_type=[jax.ShapeDtypeStruct((E_CH,), jnp.float32)] * 3,
            mesh=mesh,
            compiler_params=cp,
            scratch_types=[
                pltpu.VMEM((N_NODES,), jnp.float32),
                pltpu.VMEM((N_NODES,), jnp.float32),
                pltpu.VMEM((GW,), jnp.int32),
                pltpu.VMEM((GW,), jnp.int32),
                pltpu.VMEM((GW,), jnp.float32),
                pltpu.VMEM((GW,), jnp.float32),
            ],
        )(pos_x, pos_y, pos_z, src_c, dst_c)

    nbe = E_CH // BE

    def tc_edge(rx, ry, rz):
        rx3 = rx.reshape(nbe, 1, BE)
        ry3 = ry.reshape(nbe, 1, BE)
        rz3 = rz.reshape(nbe, 1, BE)
        return pl.pallas_call(
            _tc_edge_body,
            grid=(nbe,),
            in_specs=[
                pl.BlockSpec((1, 1, BE), lambda i: (i, 0, 0)),
                pl.BlockSpec((1, 1, BE), lambda i: (i, 0, 0)),
                pl.BlockSpec((1, 1, BE), lambda i: (i, 0, 0)),
                _full((NB, 1)),
                _full((NB, 1)),
                _full((H, NB + 9)),
                _full((H, 1)),
                _full((H, H)),
                _full((1, H)),
            ],
            out_specs=pl.BlockSpec((2, BE // 4, 128), lambda i: (0, i, 0)),
            out_shape=jax.ShapeDtypeStruct((2, E_CH // 4, 128), jnp.float32),
            compiler_params=pltpu.CompilerParams(
                fuse_transposed_lhs_in_matmul=True),
        )(rx3, ry3, rz3, cen, wid, wf1, b1f, w2b, b2r)

    def sc_scatter(pf4, srcs_c):
        pf3 = pf4.reshape(2, E_CH, 32)
        return pl.kernel(
            _sc_scatter_body,
            out_type=jax.ShapeDtypeStruct((2, N_OUT, 32), jnp.float32),
            mesh=mesh,
            compiler_params=cp,
            scratch_types=[
                pltpu.VMEM_SHARED((N_ACC, 32), jnp.float32),
                pltpu.VMEM((SKJ, 128), jnp.int32),
                pltpu.VMEM((SK, 32), jnp.float32),
            ],
        )(pf3, srcs_c, zeros_acc)

    rows_ch = E_CH // 128
    aggs = []
    rs = [sc_gather(src_g[c * E_CH:(c + 1) * E_CH],
                    dst_g[c * E_CH:(c + 1) * E_CH]) for c in range(NCH)]
    pfs = [tc_edge(*rs[c]) for c in range(NCH)]
    for c in range(NCH):
        aggs.append(sc_scatter(pfs[c], src_s[c * rows_ch:(c + 1) * rows_ch]))

    agg4a = aggs[0].reshape(2, N_OUT // 4, 128)
    agg4b = aggs[1].reshape(2, N_OUT // 4, 128)

    out_pad = pl.pallas_call(
        _tc_node_body,
        grid=(N_OUT // BN,),
        in_specs=[
            pl.BlockSpec((3, BN), lambda i: (0, i)),
            _full((3, 1)),
            _full((NB, 1)),
            _full((NB, 1)),
            _full((H, NB + 9)),
            _full((H, 1)),
            pl.BlockSpec((2, BN // 4, 128), lambda i: (0, i, 0)),
            pl.BlockSpec((2, BN // 4, 128), lambda i: (0, i, 0)),
            _full((32, OUT)),
            _full((32, OUT)),
            _full((H, OUT)),
            _full((1, OUT)),
            _full((1, OUT)),
            _full((1, OUT)),
        ],
        out_specs=pl.BlockSpec((BN, OUT), lambda i: (i, 0)),
        out_shape=jax.ShapeDtypeStruct((N_OUT, OUT), jnp.float32),
        compiler_params=pltpu.CompilerParams(
            fuse_transposed_lhs_in_matmul=True),
    )(posTp, zinc_col, cen, wid, wfz, bzf, agg4a, agg4b,
      wn0, wn1, wn2, bnr, gr, br)

    out = (out_pad.reshape(N_OUT // BN, 4, BN // 4, OUT)
           .swapaxes(1, 2)
           .reshape(N_OUT, OUT))
    return out[:N_NODES]


# speedup vs baseline: 6.6786x; 1.0562x over previous
import dataclasses
import math

import jax
import jax.numpy as jnp
from jax import lax
from jax.experimental import pallas as pl
from jax.experimental.pallas import tpu as pltpu
from jax.experimental.pallas import tpu_sc as plsc

N_NODES = 50000
NB = 16
H = 64
OUT = 128

NCH = 2
E_PAD = 819200
E_CH = E_PAD // NCH
ECT = E_CH // 16
GW = 1600
SK = 256
SKJ = SK // 128
BE = 2048
N_OUT = 51200
N_ACC = N_OUT
BN = 2048

_C0 = 1.0 / math.sqrt(4 * math.pi)
_C1 = math.sqrt(3 / (4 * math.pi))
_C2 = math.sqrt(15 / (4 * math.pi))
_C3 = math.sqrt(5 / (16 * math.pi))
_C4 = math.sqrt(15 / (16 * math.pi))


def _sc_compiler_params():
    cp = pltpu.CompilerParams(use_tc_tiling_on_sc=False)
    if "needs_layout_passes" in pltpu.CompilerParams.__dataclass_fields__:
        cp = dataclasses.replace(cp, needs_layout_passes=False)
    return cp


def _gather_windows(base, src_hbm, dst_hbm, idx_s, idx_d, comps):
    @pl.loop(0, ECT, step=GW)
    def _window(off):
        o = pl.multiple_of(base + off, GW)
        pltpu.sync_copy(src_hbm.at[pl.ds(o, GW)], idx_s)
        pltpu.sync_copy(dst_hbm.at[pl.ds(o, GW)], idx_d)

        @pl.loop(0, GW, step=16)
        def _grp(g):
            isv = idx_s[pl.ds(g, 16)]
            idv = idx_d[pl.ds(g, 16)]
            for tab, ow, _ in comps:
                ow[pl.ds(g, 16)] = (plsc.load_gather(tab, [idv])
                                    - plsc.load_gather(tab, [isv]))

        for _, ow, out_hbm in comps:
            pltpu.sync_copy(ow, out_hbm.at[pl.ds(o, GW)])


def _sc_gather_body(px_hbm, py_hbm, pz_hbm, src_hbm, dst_hbm,
                    rx_hbm, ry_hbm, rz_hbm,
                    tab0, tab1, idx_s, idx_d, o0, o1):
    c = lax.axis_index("c")
    s = lax.axis_index("s")
    base = s * ECT

    @pl.when(c == 0)
    def _():
        pltpu.sync_copy(px_hbm, tab0)
        pltpu.sync_copy(py_hbm, tab1)
        _gather_windows(base, src_hbm, dst_hbm, idx_s, idx_d,
                        [(tab0, o0, rx_hbm), (tab1, o1, ry_hbm)])

    @pl.when(c == 1)
    def _():
        pltpu.sync_copy(pz_hbm, tab0)
        _gather_windows(base, src_hbm, dst_hbm, idx_s, idx_d,
                        [(tab0, o0, rz_hbm)])


def _sc_scatter_body(pf_hbm, srcs_hbm, zero_hbm, agg_hbm, acc,
                     idx0, idx1, pfb0, pfb1, si0, sp0, si1, sp1):
    c = lax.axis_index("c")
    s = lax.axis_index("s")
    zrows = N_ACC // 16
    zoff = pl.multiple_of(s * zrows, 8)
    pltpu.sync_copy(zero_hbm.at[pl.ds(zoff, zrows)],
                    acc.at[pl.ds(zoff, zrows)])
    plsc.subcore_barrier()

    base = s * ECT

    def start(off, idxb, pfb, semi, semp):
        o = pl.multiple_of(off, SK)
        pltpu.async_copy(
            srcs_hbm.at[pl.ds(pl.multiple_of(o // 128, SKJ), SKJ)],
            idxb, semi)
        pltpu.async_copy(pf_hbm.at[c].at[pl.ds(o, SK)], pfb, semp)

    def wait(idxb, pfb, semi, semp):
        pltpu.make_async_copy(srcs_hbm.at[pl.ds(0, SKJ)], idxb, semi).wait()
        pltpu.make_async_copy(pf_hbm.at[c].at[pl.ds(0, SK)], pfb, semp).wait()

    def scatter(idxb, pfb):
        for j in range(SKJ):
            pltpu.sync_copy(pfb.at[pl.ds(j * 128, 128)],
                            acc.at[idxb.at[j]], add=True)

    start(base, idx0, pfb0, si0, sp0)

    @pl.loop(0, ECT, step=2 * SK)
    def _pair(off):
        o = base + off
        start(o + SK, idx1, pfb1, si1, sp1)
        wait(idx0, pfb0, si0, sp0)
        scatter(idx0, pfb0)

        @pl.when(off + 2 * SK < ECT)
        def _():
            start(o + 2 * SK, idx0, pfb0, si0, sp0)

        wait(idx1, pfb1, si1, sp1)
        scatter(idx1, pfb1)

    plsc.subcore_barrier()
    orows = N_OUT // 16
    ooff = pl.multiple_of(s * orows, 8)
    pltpu.sync_copy(acc.at[pl.ds(ooff, orows)],
                    agg_hbm.at[c].at[pl.ds(ooff, orows)])


def _silu(x):
    return x * (1.0 / (1.0 + jnp.exp(-x)))


def _hidden(rx, ry, rz, cen, wid, wf, bf):
    d2 = rx * rx + ry * ry + rz * rz
    d = jnp.sqrt(d2)
    dc = jnp.maximum(d, 1e-6)
    inv = 1.0 / dc
    ux, uy, uz = rx * inv, ry * inv, rz * inv
    one = jnp.ones_like(ux)
    rbf = jnp.exp(-jnp.abs(wid) * (dc - cen) ** 2)
    g = jnp.concatenate([
        rbf,
        _C0 * one,
        _C1 * uy,
        _C1 * uz,
        _C1 * ux,
        _C2 * ux * uy,
        _C2 * uy * uz,
        _C3 * (2.0 * uz * uz - ux * ux - uy * uy),
        _C2 * ux * uz,
        _C4 * (ux * ux - uy * uy),
    ], axis=0).astype(jnp.bfloat16)
    h = jnp.dot(wf, g, preferred_element_type=jnp.float32) + bf
    return _silu(h)


def _dotT(a, b):
    return jax.lax.dot_general(a, b, (((0,), (0,)), ((), ())),
                               preferred_element_type=jnp.float32)


def _tc_edge_body(rx_ref, ry_ref, rz_ref, cen_ref, wid_ref,
                  wf1_ref, b1f_ref, w2_ref, b2_ref, out_ref):
    rx = rx_ref[...].reshape(1, BE)
    ry = ry_ref[...].reshape(1, BE)
    rz = rz_ref[...].reshape(1, BE)
    h1 = _hidden(rx, ry, rz, cen_ref[...], wid_ref[...],
                 wf1_ref[...], b1f_ref[...])
    pf = _dotT(h1.astype(jnp.bfloat16), w2_ref[...])
    pf = pf + b2_ref[...]
    q = BE // 4
    out_ref[0] = jnp.concatenate(
        [pf[k * q:(k + 1) * q, :32] for k in range(4)], axis=1)
    out_ref[1] = jnp.concatenate(
        [pf[k * q:(k + 1) * q, 32:] for k in range(4)], axis=1)


def _tc_node_body(posT_ref, zinc_ref, cen_ref, wid_ref,
                  wfz_ref, bzf_ref, agga_ref, aggb_ref, wn0_ref, wn1_ref,
                  wn2_ref, bn_ref, g_ref, b_ref, out_ref):
    p = posT_ref[...] - zinc_ref[...]
    zn = _hidden(p[0:1], p[1:2], p[2:3], cen_ref[...], wid_ref[...],
                 wfz_ref[...], bzf_ref[...])
    def unpack(ref, half):
        return jnp.concatenate(
            [ref[half][:, 32 * k:32 * (k + 1)] for k in range(4)], axis=0)

    a0 = unpack(agga_ref, 0) + unpack(aggb_ref, 0)
    a1 = unpack(agga_ref, 1) + unpack(aggb_ref, 1)
    h = (jnp.dot(a0, wn0_ref[...], preferred_element_type=jnp.float32)
         + jnp.dot(a1, wn1_ref[...], preferred_element_type=jnp.float32)
         + _dotT(zn, wn2_ref[...])
         + bn_ref[...])
    m = jnp.mean(h, axis=-1, keepdims=True)
    hc = h - m
    v = jnp.mean(hc * hc, axis=-1, keepdims=True)
    y = hc / jnp.sqrt(v + 1e-5) * g_ref[...] + b_ref[...]
    out_ref[...] = _silu(y)


def _full(shape):
    return pl.BlockSpec(shape, lambda i: tuple(0 for _ in shape))


def kernel(pos, zinc_pos, edge_index, rbf_centers, rbf_widths, sh_w, sh_b,
           w1, b1, w2, b2, wz, bz, wn, bn, gamma, beta):
    E = edge_index.shape[1]
    pad = E_PAD - E
    src = edge_index[0].astype(jnp.int32)
    dst = edge_index[1].astype(jnp.int32)
    src_g = jnp.concatenate([src, jnp.zeros((pad,), jnp.int32)])
    dst_g = jnp.concatenate([dst, jnp.zeros((pad,), jnp.int32)])
    dummy = N_NODES + (jnp.arange(pad, dtype=jnp.int32) % 1024)
    src_s = (jnp.concatenate([src, dummy])
             .reshape(E_PAD // BE, 4, BE // 4)
             .swapaxes(1, 2)
             .reshape(-1, 128))
    pos_x = pos[:, 0]
    pos_y = pos[:, 1]
    pos_z = pos[:, 2]
    posT = jnp.pad(pos.T, ((0, 0), (0, N_OUT - N_NODES)))
    posTp = (posT.reshape(3, N_OUT // BN, BN // 4, 4)
             .swapaxes(2, 3)
             .reshape(3, N_OUT))

    cen = rbf_centers.reshape(NB, 1)
    wid = rbf_widths.reshape(NB, 1)
    bf16 = jnp.bfloat16
    wf1 = jnp.concatenate([w1[:NB], sh_w @ w1[NB:]], axis=0).T.astype(bf16)
    b1f = (b1 + sh_b @ w1[NB:]).reshape(H, 1)
    wfz = jnp.concatenate([wz[:NB], sh_w @ wz[NB:]], axis=0).T.astype(bf16)
    bzf = (bz + sh_b @ wz[NB:]).reshape(H, 1)
    w2b = w2.astype(bf16)
    b2r = b2.reshape(1, H)
    wn0 = wn[:32]
    wn1 = wn[32:64]
    wn2 = wn[64:]
    bnr = bn.reshape(1, OUT)
    gr = gamma.reshape(1, OUT)
    br = beta.reshape(1, OUT)
    zinc_col = zinc_pos.reshape(3, 1)
    zeros_acc = jnp.zeros((N_ACC, 32), jnp.float32)

    mesh = plsc.VectorSubcoreMesh(core_axis_name="c", subcore_axis_name="s")
    cp = _sc_compiler_params()

    def sc_gather(src_c, dst_c):
        return pl.kernel(
            _sc_gather_body,
            out_type=[jax.ShapeDtypeStruct((E_CH,), jnp.float32)] * 3,
            mesh=mesh,
            compiler_params=cp,
            scratch_types=[
                pltpu.VMEM((N_NODES,), jnp.float32),
                pltpu.VMEM((N_NODES,), jnp.float32),
                pltpu.VMEM((GW,), jnp.int32),
                pltpu.VMEM((GW,), jnp.int32),
                pltpu.VMEM((GW,), jnp.float32),
                pltpu.VMEM((GW,), jnp.float32),
            ],
        )(pos_x, pos_y, pos_z, src_c, dst_c)

    nbe = E_CH // BE

    def tc_edge(rx, ry, rz):
        rx3 = rx.reshape(nbe, 1, BE)
        ry3 = ry.reshape(nbe, 1, BE)
        rz3 = rz.reshape(nbe, 1, BE)
        return pl.pallas_call(
            _tc_edge_body,
            grid=(nbe,),
            in_specs=[
                pl.BlockSpec((1, 1, BE), lambda i: (i, 0, 0)),
                pl.BlockSpec((1, 1, BE), lambda i: (i, 0, 0)),
                pl.BlockSpec((1, 1, BE), lambda i: (i, 0, 0)),
                _full((NB, 1)),
                _full((NB, 1)),
                _full((H, NB + 9)),
                _full((H, 1)),
                _full((H, H)),
                _full((1, H)),
            ],
            out_specs=pl.BlockSpec((2, BE // 4, 128), lambda i: (0, i, 0)),
            out_shape=jax.ShapeDtypeStruct((2, E_CH // 4, 128), jnp.float32),
            compiler_params=pltpu.CompilerParams(
                fuse_transposed_lhs_in_matmul=True),
        )(rx3, ry3, rz3, cen, wid, wf1, b1f, w2b, b2r)

    def sc_scatter(pf4, srcs_c):
        pf3 = pf4.reshape(2, E_CH, 32)
        return pl.kernel(
            _sc_scatter_body,
            out_type=jax.ShapeDtypeStruct((2, N_OUT, 32), jnp.float32),
            mesh=mesh,
            compiler_params=cp,
            scratch_types=[
                pltpu.VMEM_SHARED((N_ACC, 32), jnp.float32),
                pltpu.VMEM((SKJ, 128), jnp.int32),
                pltpu.VMEM((SKJ, 128), jnp.int32),
                pltpu.VMEM((SK, 32), jnp.float32),
                pltpu.VMEM((SK, 32), jnp.float32),
                pltpu.SemaphoreType.DMA,
                pltpu.SemaphoreType.DMA,
                pltpu.SemaphoreType.DMA,
                pltpu.SemaphoreType.DMA,
            ],
        )(pf3, srcs_c, zeros_acc)

    rows_ch = E_CH // 128
    aggs = []
    rs = [sc_gather(src_g[c * E_CH:(c + 1) * E_CH],
                    dst_g[c * E_CH:(c + 1) * E_CH]) for c in range(NCH)]
    pfs = [tc_edge(*rs[c]) for c in range(NCH)]
    for c in range(NCH):
        aggs.append(sc_scatter(pfs[c], src_s[c * rows_ch:(c + 1) * rows_ch]))

    agg4a = aggs[0].reshape(2, N_OUT // 4, 128)
    agg4b = aggs[1].reshape(2, N_OUT // 4, 128)

    out_pad = pl.pallas_call(
        _tc_node_body,
        grid=(N_OUT // BN,),
        in_specs=[
            pl.BlockSpec((3, BN), lambda i: (0, i)),
            _full((3, 1)),
            _full((NB, 1)),
            _full((NB, 1)),
            _full((H, NB + 9)),
            _full((H, 1)),
            pl.BlockSpec((2, BN // 4, 128), lambda i: (0, i, 0)),
            pl.BlockSpec((2, BN // 4, 128), lambda i: (0, i, 0)),
            _full((32, OUT)),
            _full((32, OUT)),
            _full((H, OUT)),
            _full((1, OUT)),
            _full((1, OUT)),
            _full((1, OUT)),
        ],
        out_specs=pl.BlockSpec((BN, OUT), lambda i: (i, 0)),
        out_shape=jax.ShapeDtypeStruct((N_OUT, OUT), jnp.float32),
        compiler_params=pltpu.CompilerParams(
            fuse_transposed_lhs_in_matmul=True),
    )(posTp, zinc_col, cen, wid, wfz, bzf, agg4a, agg4b,
      wn0, wn1, wn2, bnr, gr, br)

    out = (out_pad.reshape(N_OUT // BN, 4, BN // 4, OUT)
           .swapaxes(1, 2)
           .reshape(N_OUT, OUT))
    return out[:N_NODES]


# speedup vs baseline: 6.9084x; 1.0344x over previous
import dataclasses
import math

import jax
import jax.numpy as jnp
from jax import lax
from jax.experimental import pallas as pl
from jax.experimental.pallas import tpu as pltpu
from jax.experimental.pallas import tpu_sc as plsc

N_NODES = 50000
NB = 16
H = 64
OUT = 128

NCH = 2
E_PAD = 819200
E_CH = E_PAD // NCH
ECT = E_CH // 16
GW = 6400
SK = 256
SKJ = SK // 128
BE = 2048
N_OUT = 51200
N_ACC = N_OUT
BN = 2048

_C0 = 1.0 / math.sqrt(4 * math.pi)
_C1 = math.sqrt(3 / (4 * math.pi))
_C2 = math.sqrt(15 / (4 * math.pi))
_C3 = math.sqrt(5 / (16 * math.pi))
_C4 = math.sqrt(15 / (16 * math.pi))


def _sc_compiler_params():
    cp = pltpu.CompilerParams(use_tc_tiling_on_sc=False)
    if "needs_layout_passes" in pltpu.CompilerParams.__dataclass_fields__:
        cp = dataclasses.replace(cp, needs_layout_passes=False)
    return cp


def _gather_windows(base, src_hbm, dst_hbm, idx_s, idx_d, comps):
    @pl.loop(0, ECT, step=GW)
    def _window(off):
        o = pl.multiple_of(base + off, GW)
        pltpu.sync_copy(src_hbm.at[pl.ds(o, GW)], idx_s)
        pltpu.sync_copy(dst_hbm.at[pl.ds(o, GW)], idx_d)

        @pl.loop(0, GW, step=16)
        def _grp(g):
            isv = idx_s[pl.ds(g, 16)]
            idv = idx_d[pl.ds(g, 16)]
            for tab, ow, _ in comps:
                ow[pl.ds(g, 16)] = (plsc.load_gather(tab, [idv])
                                    - plsc.load_gather(tab, [isv]))

        for _, ow, out_hbm in comps:
            pltpu.sync_copy(ow, out_hbm.at[pl.ds(o, GW)])


def _sc_gather_body(px_hbm, py_hbm, pz_hbm, src_hbm, dst_hbm,
                    rx_hbm, ry_hbm, rz_hbm,
                    tab0, tab1, idx_s, idx_d, o0, o1):
    c = lax.axis_index("c")
    s = lax.axis_index("s")
    base = s * ECT

    @pl.when(c == 0)
    def _():
        pltpu.sync_copy(px_hbm, tab0)
        pltpu.sync_copy(py_hbm, tab1)
        _gather_windows(base, src_hbm, dst_hbm, idx_s, idx_d,
                        [(tab0, o0, rx_hbm), (tab1, o1, ry_hbm)])

    @pl.when(c == 1)
    def _():
        pltpu.sync_copy(pz_hbm, tab0)
        _gather_windows(base, src_hbm, dst_hbm, idx_s, idx_d,
                        [(tab0, o0, rz_hbm)])


def _sc_scatter_body(pf_hbm, srcs_hbm, zero_hbm, agg_hbm, acc,
                     idx0, idx1, pfb0, pfb1, si0, sp0, si1, sp1):
    c = lax.axis_index("c")
    s = lax.axis_index("s")
    zrows = N_ACC // 16
    zoff = pl.multiple_of(s * zrows, 8)
    pltpu.sync_copy(zero_hbm.at[pl.ds(zoff, zrows)],
                    acc.at[pl.ds(zoff, zrows)])
    plsc.subcore_barrier()

    base = s * ECT

    def start(off, idxb, pfb, semi, semp):
        o = pl.multiple_of(off, SK)
        pltpu.async_copy(
            srcs_hbm.at[pl.ds(pl.multiple_of(o // 128, SKJ), SKJ)],
            idxb, semi)
        pltpu.async_copy(pf_hbm.at[c].at[pl.ds(o, SK)], pfb, semp)

    def wait(idxb, pfb, semi, semp):
        pltpu.make_async_copy(srcs_hbm.at[pl.ds(0, SKJ)], idxb, semi).wait()
        pltpu.make_async_copy(pf_hbm.at[c].at[pl.ds(0, SK)], pfb, semp).wait()

    def scatter(idxb, pfb):
        for j in range(SKJ):
            pltpu.sync_copy(pfb.at[pl.ds(j * 128, 128)],
                            acc.at[idxb.at[j]], add=True)

    start(base, idx0, pfb0, si0, sp0)

    @pl.loop(0, ECT, step=2 * SK)
    def _pair(off):
        o = base + off
        start(o + SK, idx1, pfb1, si1, sp1)
        wait(idx0, pfb0, si0, sp0)
        scatter(idx0, pfb0)

        @pl.when(off + 2 * SK < ECT)
        def _():
            start(o + 2 * SK, idx0, pfb0, si0, sp0)

        wait(idx1, pfb1, si1, sp1)
        scatter(idx1, pfb1)

    plsc.subcore_barrier()
    orows = N_OUT // 16
    ooff = pl.multiple_of(s * orows, 8)
    pltpu.sync_copy(acc.at[pl.ds(ooff, orows)],
                    agg_hbm.at[c].at[pl.ds(ooff, orows)])


def _silu(x):
    return x * (1.0 / (1.0 + jnp.exp(-x)))


def _geom(rx, ry, rz, cen, wid):
    d2 = rx * rx + ry * ry + rz * rz
    d = jnp.sqrt(d2)
    dc = jnp.maximum(d, 1e-6)
    inv = 1.0 / dc
    ux, uy, uz = rx * inv, ry * inv, rz * inv
    one = jnp.ones_like(ux)
    rbf = jnp.exp(-jnp.abs(wid) * (dc - cen) ** 2)
    return jnp.concatenate([
        rbf,
        _C0 * one,
        _C1 * uy,
        _C1 * uz,
        _C1 * ux,
        _C2 * ux * uy,
        _C2 * uy * uz,
        _C3 * (2.0 * uz * uz - ux * ux - uy * uy),
        _C2 * ux * uz,
        _C4 * (ux * ux - uy * uy),
    ], axis=0).astype(jnp.bfloat16)


def _hidden(rx, ry, rz, cen, wid, wf, bf):
    d2 = rx * rx + ry * ry + rz * rz
    d = jnp.sqrt(d2)
    dc = jnp.maximum(d, 1e-6)
    inv = 1.0 / dc
    ux, uy, uz = rx * inv, ry * inv, rz * inv
    one = jnp.ones_like(ux)
    rbf = jnp.exp(-jnp.abs(wid) * (dc - cen) ** 2)
    g = jnp.concatenate([
        rbf,
        _C0 * one,
        _C1 * uy,
        _C1 * uz,
        _C1 * ux,
        _C2 * ux * uy,
        _C2 * uy * uz,
        _C3 * (2.0 * uz * uz - ux * ux - uy * uy),
        _C2 * ux * uz,
        _C4 * (ux * ux - uy * uy),
    ], axis=0).astype(jnp.bfloat16)
    h = jnp.dot(wf, g, preferred_element_type=jnp.float32) + bf
    return _silu(h)


def _dotT(a, b):
    return jax.lax.dot_general(a, b, (((0,), (0,)), ((), ())),
                               preferred_element_type=jnp.float32)


def _tc_edge_body(rx_ref, ry_ref, rz_ref, cen_ref, wid_ref,
                  wf1_ref, b1f_ref, w2_ref, b2_ref, out_ref):
    rx = rx_ref[...].reshape(1, BE)
    ry = ry_ref[...].reshape(1, BE)
    rz = rz_ref[...].reshape(1, BE)
    g = _geom(rx, ry, rz, cen_ref[...], wid_ref[...])
    gT = g.T
    h1 = _silu(jnp.dot(gT, wf1_ref[...].T,
                       preferred_element_type=jnp.float32) + b1f_ref[...].T)
    pf = jnp.dot(h1.astype(jnp.bfloat16), w2_ref[...],
                 preferred_element_type=jnp.float32)
    pf = pf + b2_ref[...]
    q = BE // 4
    out_ref[0] = jnp.concatenate(
        [pf[k * q:(k + 1) * q, :32] for k in range(4)], axis=1)
    out_ref[1] = jnp.concatenate(
        [pf[k * q:(k + 1) * q, 32:] for k in range(4)], axis=1)


def _tc_node_body(posT_ref, zinc_ref, cen_ref, wid_ref,
                  wfz_ref, bzf_ref, agga_ref, aggb_ref, wn0_ref, wn1_ref,
                  wn2_ref, bn_ref, g_ref, b_ref, out_ref):
    p = posT_ref[...] - zinc_ref[...]
    zn = _hidden(p[0:1], p[1:2], p[2:3], cen_ref[...], wid_ref[...],
                 wfz_ref[...], bzf_ref[...])
    def unpack(ref, half):
        return jnp.concatenate(
            [ref[half][:, 32 * k:32 * (k + 1)] for k in range(4)], axis=0)

    a0 = unpack(agga_ref, 0) + unpack(aggb_ref, 0)
    a1 = unpack(agga_ref, 1) + unpack(aggb_ref, 1)
    h = (jnp.dot(a0, wn0_ref[...], preferred_element_type=jnp.float32)
         + jnp.dot(a1, wn1_ref[...], preferred_element_type=jnp.float32)
         + _dotT(zn, wn2_ref[...])
         + bn_ref[...])
    m = jnp.mean(h, axis=-1, keepdims=True)
    hc = h - m
    v = jnp.mean(hc * hc, axis=-1, keepdims=True)
    y = hc / jnp.sqrt(v + 1e-5) * g_ref[...] + b_ref[...]
    out_ref[...] = _silu(y)


def _full(shape):
    return pl.BlockSpec(shape, lambda i: tuple(0 for _ in shape))


def kernel(pos, zinc_pos, edge_index, rbf_centers, rbf_widths, sh_w, sh_b,
           w1, b1, w2, b2, wz, bz, wn, bn, gamma, beta):
    E = edge_index.shape[1]
    pad = E_PAD - E
    src = edge_index[0].astype(jnp.int32)
    dst = edge_index[1].astype(jnp.int32)
    src_g = jnp.concatenate([src, jnp.zeros((pad,), jnp.int32)])
    dst_g = jnp.concatenate([dst, jnp.zeros((pad,), jnp.int32)])
    dummy = N_NODES + (jnp.arange(pad, dtype=jnp.int32) % 1024)
    src_s = (jnp.concatenate([src, dummy])
             .reshape(E_PAD // BE, 4, BE // 4)
             .swapaxes(1, 2)
             .reshape(-1, 128))
    pos_x = pos[:, 0]
    pos_y = pos[:, 1]
    pos_z = pos[:, 2]
    posT = jnp.pad(pos.T, ((0, 0), (0, N_OUT - N_NODES)))
    posTp = (posT.reshape(3, N_OUT // BN, BN // 4, 4)
             .swapaxes(2, 3)
             .reshape(3, N_OUT))

    cen = rbf_centers.reshape(NB, 1)
    wid = rbf_widths.reshape(NB, 1)
    bf16 = jnp.bfloat16
    wf1 = jnp.concatenate([w1[:NB], sh_w @ w1[NB:]], axis=0).T.astype(bf16)
    b1f = (b1 + sh_b @ w1[NB:]).reshape(H, 1)
    wfz = jnp.concatenate([wz[:NB], sh_w @ wz[NB:]], axis=0).T.astype(bf16)
    bzf = (bz + sh_b @ wz[NB:]).reshape(H, 1)
    w2b = w2.astype(bf16)
    b2r = b2.reshape(1, H)
    wn0 = wn[:32]
    wn1 = wn[32:64]
    wn2 = wn[64:]
    bnr = bn.reshape(1, OUT)
    gr = gamma.reshape(1, OUT)
    br = beta.reshape(1, OUT)
    zinc_col = zinc_pos.reshape(3, 1)
    zeros_acc = jnp.zeros((N_ACC, 32), jnp.float32)

    mesh = plsc.VectorSubcoreMesh(core_axis_name="c", subcore_axis_name="s")
    cp = _sc_compiler_params()

    def sc_gather(src_c, dst_c):
        return pl.kernel(
            _sc_gather_body,
            out_type=[jax.ShapeDtypeStruct((E_CH,), jnp.float32)] * 3,
            mesh=mesh,
            compiler_params=cp,
            scratch_types=[
                pltpu.VMEM((N_NODES,), jnp.float32),
                pltpu.VMEM((N_NODES,), jnp.float32),
                pltpu.VMEM((GW,), jnp.int32),
                pltpu.VMEM((GW,), jnp.int32),
                pltpu.VMEM((GW,), jnp.float32),
                pltpu.VMEM((GW,), jnp.float32),
            ],
        )(pos_x, pos_y, pos_z, src_c, dst_c)

    nbe = E_CH // BE

    def tc_edge(rx, ry, rz):
        rx3 = rx.reshape(nbe, 1, BE)
        ry3 = ry.reshape(nbe, 1, BE)
        rz3 = rz.reshape(nbe, 1, BE)
        return pl.pallas_call(
            _tc_edge_body,
            grid=(nbe,),
            in_specs=[
                pl.BlockSpec((1, 1, BE), lambda i: (i, 0, 0)),
                pl.BlockSpec((1, 1, BE), lambda i: (i, 0, 0)),
                pl.BlockSpec((1, 1, BE), lambda i: (i, 0, 0)),
                _full((NB, 1)),
                _full((NB, 1)),
                _full((H, NB + 9)),
                _full((H, 1)),
                _full((H, H)),
                _full((1, H)),
            ],
            out_specs=pl.BlockSpec((2, BE // 4, 128), lambda i: (0, i, 0)),
            out_shape=jax.ShapeDtypeStruct((2, E_CH // 4, 128), jnp.float32),
            compiler_params=pltpu.CompilerParams(
                fuse_transposed_lhs_in_matmul=True),
        )(rx3, ry3, rz3, cen, wid, wf1, b1f, w2b, b2r)

    def sc_scatter(pf4, srcs_c):
        pf3 = pf4.reshape(2, E_CH, 32)
        return pl.kernel(
            _sc_scatter_body,
            out_type=jax.ShapeDtypeStruct((2, N_OUT, 32), jnp.float32),
            mesh=mesh,
            compiler_params=cp,
            scratch_types=[
                pltpu.VMEM_SHARED((N_ACC, 32), jnp.float32),
                pltpu.VMEM((SKJ, 128), jnp.int32),
                pltpu.VMEM((SKJ, 128), jnp.int32),
                pltpu.VMEM((SK, 32), jnp.float32),
                pltpu.VMEM((SK, 32), jnp.float32),
                pltpu.SemaphoreType.DMA,
                pltpu.SemaphoreType.DMA,
                pltpu.SemaphoreType.DMA,
                pltpu.SemaphoreType.DMA,
            ],
        )(pf3, srcs_c, zeros_acc)

    rows_ch = E_CH // 128
    aggs = []
    rs = [sc_gather(src_g[c * E_CH:(c + 1) * E_CH],
                    dst_g[c * E_CH:(c + 1) * E_CH]) for c in range(NCH)]
    pfs = [tc_edge(*rs[c]) for c in range(NCH)]
    for c in range(NCH):
        aggs.append(sc_scatter(pfs[c], src_s[c * rows_ch:(c + 1) * rows_ch]))

    agg4a = aggs[0].reshape(2, N_OUT // 4, 128)
    agg4b = aggs[1].reshape(2, N_OUT // 4, 128)

    out_pad = pl.pallas_call(
        _tc_node_body,
        grid=(N_OUT // BN,),
        in_specs=[
            pl.BlockSpec((3, BN), lambda i: (0, i)),
            _full((3, 1)),
            _full((NB, 1)),
            _full((NB, 1)),
            _full((H, NB + 9)),
            _full((H, 1)),
            pl.BlockSpec((2, BN // 4, 128), lambda i: (0, i, 0)),
            pl.BlockSpec((2, BN // 4, 128), lambda i: (0, i, 0)),
            _full((32, OUT)),
            _full((32, OUT)),
            _full((H, OUT)),
            _full((1, OUT)),
            _full((1, OUT)),
            _full((1, OUT)),
        ],
        out_specs=pl.BlockSpec((BN, OUT), lambda i: (i, 0)),
        out_shape=jax.ShapeDtypeStruct((N_OUT, OUT), jnp.float32),
        compiler_params=pltpu.CompilerParams(
            fuse_transposed_lhs_in_matmul=True),
    )(posTp, zinc_col, cen, wid, wfz, bzf, agg4a, agg4b,
      wn0, wn1, wn2, bnr, gr, br)

    out = (out_pad.reshape(N_OUT // BN, 4, BN // 4, OUT)
           .swapaxes(1, 2)
           .reshape(N_OUT, OUT))
    return out[:N_NODES]
